# baseline (speedup 1.0000x reference)
import functools

import jax
import jax.numpy as jnp
from jax import lax
from jax.experimental import pallas as pl
from jax.experimental.pallas import tpu as pltpu
from jax.experimental.pallas import tpu_sc as plsc

NUM_EMBEDDINGS = 1000000
EMBEDDING_DIM = 32
BATCH = 16384
FIELDS = 26

_B = BATCH * FIELDS
_CHUNK = 128
_NB = 8
_LEAD = 4
_NW = 32
_NCH = _B // (_NW * _CHUNK)
_NT = (NUM_EMBEDDINGS + _CHUNK - 1) // _CHUNK
_TAIL = NUM_EMBEDDINGS - (_NT - 1) * _CHUNK
_TW = (_NT - 1) % _NW


def _make_fmt():
    mesh = plsc.VectorSubcoreMesh(core_axis_name="c", subcore_axis_name="s")

    @functools.partial(
        pl.kernel,
        mesh=mesh,
        out_type=jax.ShapeDtypeStruct(
            (NUM_EMBEDDINGS * EMBEDDING_DIM // _CHUNK, _CHUNK), jnp.float32
        ),
        scratch_types=[
            pltpu.VMEM((8 * EMBEDDING_DIM, _CHUNK), jnp.float32),
            pltpu.VMEM((8 * EMBEDDING_DIM, _CHUNK), jnp.float32),
            pltpu.SemaphoreType.DMA((8,)),
            pltpu.SemaphoreType.DMA((8,)),
        ],
        compiler_params=pltpu.CompilerParams(
            use_tc_tiling_on_sc=True, needs_layout_passes=False
        ),
    )
    def k(wt_hbm, wtail_hbm, out_hbm, buf, obuf, gsem, osem):
        wid = lax.axis_index("s") * 2 + lax.axis_index("c")
        nt = 244 + jnp.where(wid < _NT % _NW, 1, 0)
        lane = lax.iota(jnp.int32, 16)
        r32 = lane * EMBEDDING_DIM

        def in_start(t, b):
            for g in range(4):
                pltpu.async_copy(
                    wt_hbm.at[pl.ds(8 * g, 8), pl.ds(t * _CHUNK, _CHUNK)],
                    buf.at[pl.ds(b * EMBEDDING_DIM + 8 * g, 8)],
                    gsem.at[b],
                )

        def in_start_tail(b):
            for g in range(4):
                pltpu.async_copy(
                    wtail_hbm.at[pl.ds(8 * g, 8)],
                    buf.at[pl.ds(b * EMBEDDING_DIM + 8 * g, 8)],
                    gsem.at[b],
                )

        def in_wait(b):
            for g in range(4):
                pltpu.make_async_copy(
                    wt_hbm.at[pl.ds(8 * g, 8), pl.ds(0, _CHUNK)],
                    buf.at[pl.ds(b * EMBEDDING_DIM + 8 * g, 8)],
                    gsem.at[b],
                ).wait()

        def transpose(b, nr):
            boff = b * EMBEDDING_DIM
            rr = range(0, nr, 16)
            l8 = [lane + r0 for r0 in rr]
            r8 = [((lane + r0) >> 2) + boff for r0 in rr]
            m8 = [((lane + r0) & 3) * EMBEDDING_DIM for r0 in rr]

            @plsc.parallel_loop(0, EMBEDDING_DIM, unroll=4)
            def col(c0):
                ci = (lane + c0) & (EMBEDDING_DIM - 1)
                cib = ci + boff
                for i in range(len(l8)):
                    v = plsc.load_gather(buf, [cib, l8[i]])
                    plsc.store_scatter(obuf, [r8[i], m8[i] + ci], v)

        def out_start(t, b, nrows):
            pltpu.async_copy(
                obuf.at[pl.ds(b * EMBEDDING_DIM, nrows)],
                out_hbm.at[pl.ds(t * EMBEDDING_DIM, nrows)],
                osem.at[b],
            )

        def out_wait(b, nrows):
            pltpu.make_async_copy(
                obuf.at[pl.ds(b * EMBEDDING_DIM, nrows)],
                out_hbm.at[pl.ds(0, nrows)],
                osem.at[b],
            ).wait()

        def t_of(kk):
            return wid + _NW * kk

        def body_one(kk, b):
            t = t_of(kk)
            is_tail = jnp.logical_and(wid == _TW, t == _NT - 1)

            @pl.when(kk + 7 < nt)
            def _():
                tn = t_of(kk + 7)

                @pl.when(jnp.logical_or(wid != _TW, tn != _NT - 1))
                def _():
                    in_start(tn, (b + 7) % 8)

                @pl.when(jnp.logical_and(wid == _TW, tn == _NT - 1))
                def _():
                    in_start_tail((b + 7) % 8)

            in_wait(b)

            @pl.when(kk >= 8)
            def _():
                out_wait(b, EMBEDDING_DIM)

            transpose(b, _CHUNK)

            @pl.when(jnp.logical_not(is_tail))
            def _():
                out_start(t, b, EMBEDDING_DIM)

            @pl.when(is_tail)
            def _():
                out_start(t, b, _TAIL * EMBEDDING_DIM // _CHUNK)

        for kk0 in range(7):
            in_start(t_of(kk0), kk0)

        def oct_(k8, carry):
            for bb in range(8):
                kk = 8 * k8 + bb

                @pl.when(kk < nt)
                def _():
                    body_one(kk, bb)
            return carry

        lax.fori_loop(0, (244 + 1 + 7) // 8, oct_, 0)

        @pl.when(wid == _TW)
        def _():
            for b in (5, 6, 7, 0, 1, 2, 3):
                out_wait(b, EMBEDDING_DIM)
            out_wait(4, _TAIL * EMBEDDING_DIM // _CHUNK)

        @pl.when(jnp.logical_and(wid < _NT % _NW, wid != _TW))
        def _():
            for b in (5, 6, 7, 0, 1, 2, 3, 4):
                out_wait(b, EMBEDDING_DIM)

        @pl.when(wid >= _NT % _NW)
        def _():
            for b in (4, 5, 6, 7, 0, 1, 2, 3):
                out_wait(b, EMBEDDING_DIM)

    return k


def _make_gather():
    mesh = plsc.VectorSubcoreMesh(core_axis_name="c", subcore_axis_name="s")

    @functools.partial(
        pl.kernel,
        mesh=mesh,
        out_type=jax.ShapeDtypeStruct(
            (FIELDS, EMBEDDING_DIM // 8, BATCH // _CHUNK, 8, _CHUNK), jnp.float32
        ),
        scratch_types=[
            pltpu.VMEM((_NCH, _CHUNK), jnp.int32),
            pltpu.VMEM((_NB * _CHUNK, EMBEDDING_DIM), jnp.float32),
            pltpu.VMEM((_NB, EMBEDDING_DIM, _CHUNK + 1), jnp.float32),
            pltpu.SemaphoreType.DMA((_NB,)),
            pltpu.SemaphoreType.DMA((_NB,)),
        ],
        compiler_params=pltpu.CompilerParams(
            use_tc_tiling_on_sc=False, needs_layout_passes=False
        ),
    )
    def k(idx_hbm, table_hbm, out_hbm, idx_v, rows_v, tr_v, gsem, osem):
        wid = lax.axis_index("s") * 2 + lax.axis_index("c")
        ch0 = wid * _NCH
        pltpu.sync_copy(idx_hbm.at[wid], idx_v)

        def gather_start(c, b):
            pltpu.async_copy(
                table_hbm.at[idx_v.at[c]],
                rows_v.at[pl.ds(b * _CHUNK, _CHUNK)],
                gsem.at[b],
            )

        def gather_wait(b):
            pltpu.make_async_copy(
                table_hbm.at[idx_v.at[0]],
                rows_v.at[pl.ds(b * _CHUNK, _CHUNK)],
                gsem.at[b],
            ).wait()

        def out_start(j, b):
            ch = ch0 + j
            f = ch >> 7
            bb = ch & 127
            for g in range(4):
                pltpu.async_copy(
                    tr_v.at[b, pl.ds(8 * g, 8), pl.ds(0, _CHUNK)],
                    out_hbm.at[f, g, bb],
                    osem.at[b],
                )

        def out_wait(b):
            for g in range(4):
                pltpu.make_async_copy(
                    tr_v.at[b, pl.ds(8 * g, 8), pl.ds(0, _CHUNK)],
                    out_hbm.at[0, g, 0],
                    osem.at[b],
                ).wait()

        lane = lax.iota(jnp.int32, 16)
        chalf = [lane, lane + 16]

        for c in range(_LEAD):
            gather_start(c, c)

        def group(gi, carry):
            g0 = gi * _NB
            for b in range(_NB):
                j = g0 + b
                nxt = j + _LEAD

                @pl.when(nxt < _NCH)
                def _():
                    gather_start(nxt, (b + _LEAD) % _NB)

                gather_wait(b)

                @pl.when(j >= _NB)
                def _():
                    out_wait(b)

                bsp = jnp.full((16,), b, jnp.int32)

                @plsc.parallel_loop(0, _CHUNK, unroll=16)
                def _tr(r):
                    rsp = jnp.full((16,), r, jnp.int32)
                    for h in range(2):
                        v = rows_v[b * _CHUNK + r, pl.ds(16 * h, 16)]
                        plsc.store_scatter(tr_v, [bsp, chalf[h], rsp], v)

                out_start(j, b)
            return carry

        lax.fori_loop(0, _NCH // _NB, group, 0)
        for b in range(_NB):
            out_wait(b)

    return k


@jax.jit
def kernel(input, weight):
    idx = input.T.reshape(_NW, _NCH, _CHUNK)
    wt = weight.T
    wtail = jnp.pad(wt[:, _CHUNK * (_NT - 1):], ((0, 0), (0, _CHUNK - _TAIL)))
    wlin = _make_fmt()(wt, wtail)
    table = wlin.reshape(NUM_EMBEDDINGS, EMBEDDING_DIM)
    out5d = _make_gather()(idx, table)
    return out5d.transpose(2, 4, 0, 1, 3).reshape(BATCH, FIELDS, EMBEDDING_DIM)


# speedup vs baseline: 1.0259x; 1.0259x over previous
import functools

import jax
import jax.numpy as jnp
from jax import lax
from jax.experimental import pallas as pl
from jax.experimental.pallas import tpu as pltpu
from jax.experimental.pallas import tpu_sc as plsc

NUM_EMBEDDINGS = 1000000
EMBEDDING_DIM = 32
BATCH = 16384
FIELDS = 26

_B = BATCH * FIELDS
_CHUNK = 128
_NB = 8
_LEAD = 4
_NW = 32
_NCH = _B // (_NW * _CHUNK)
_NT = (NUM_EMBEDDINGS + _CHUNK - 1) // _CHUNK
_TAIL = NUM_EMBEDDINGS - (_NT - 1) * _CHUNK
_TW = (_NT - 1) % _NW


def _make_fmt():
    mesh = plsc.VectorSubcoreMesh(core_axis_name="c", subcore_axis_name="s")

    @functools.partial(
        pl.kernel,
        mesh=mesh,
        out_type=jax.ShapeDtypeStruct(
            (NUM_EMBEDDINGS * EMBEDDING_DIM // _CHUNK, _CHUNK), jnp.float32
        ),
        scratch_types=[
            pltpu.VMEM((8 * EMBEDDING_DIM, _CHUNK), jnp.float32),
            pltpu.VMEM((8 * EMBEDDING_DIM, _CHUNK), jnp.float32),
            pltpu.SemaphoreType.DMA((8,)),
            pltpu.SemaphoreType.DMA((8,)),
        ],
        compiler_params=pltpu.CompilerParams(
            use_tc_tiling_on_sc=True, needs_layout_passes=False
        ),
    )
    def k(wt_hbm, wtail_hbm, out_hbm, buf, obuf, gsem, osem):
        wid = lax.axis_index("s") * 2 + lax.axis_index("c")
        nt = 244 + jnp.where(wid < _NT % _NW, 1, 0)
        lane = lax.iota(jnp.int32, 16)
        r32 = lane * EMBEDDING_DIM

        def in_start(t, b):
            for g in range(4):
                pltpu.async_copy(
                    wt_hbm.at[pl.ds(8 * g, 8), pl.ds(t * _CHUNK, _CHUNK)],
                    buf.at[pl.ds(b * EMBEDDING_DIM + 8 * g, 8)],
                    gsem.at[b],
                )

        def in_start_tail(b):
            for g in range(4):
                pltpu.async_copy(
                    wtail_hbm.at[pl.ds(8 * g, 8)],
                    buf.at[pl.ds(b * EMBEDDING_DIM + 8 * g, 8)],
                    gsem.at[b],
                )

        def in_wait(b):
            for g in range(4):
                pltpu.make_async_copy(
                    wt_hbm.at[pl.ds(8 * g, 8), pl.ds(0, _CHUNK)],
                    buf.at[pl.ds(b * EMBEDDING_DIM + 8 * g, 8)],
                    gsem.at[b],
                ).wait()

        def transpose(b, nr):
            boff = b * EMBEDDING_DIM
            rr = range(0, nr, 16)
            l8 = [lane + r0 for r0 in rr]
            r8 = [((lane + r0) >> 2) + boff for r0 in rr]
            m8 = [((lane + r0) & 3) * EMBEDDING_DIM for r0 in rr]

            @plsc.parallel_loop(0, EMBEDDING_DIM, unroll=4)
            def col(c0):
                ci = (lane + c0) & (EMBEDDING_DIM - 1)
                cib = ci + boff
                for i in range(len(l8)):
                    v = plsc.load_gather(buf, [cib, l8[i]])
                    plsc.store_scatter(obuf, [r8[i], m8[i] + ci], v)

        def out_start(t, b, nrows):
            pltpu.async_copy(
                obuf.at[pl.ds(b * EMBEDDING_DIM, nrows)],
                out_hbm.at[pl.ds(t * EMBEDDING_DIM, nrows)],
                osem.at[b],
            )

        def out_wait(b, nrows):
            pltpu.make_async_copy(
                obuf.at[pl.ds(b * EMBEDDING_DIM, nrows)],
                out_hbm.at[pl.ds(0, nrows)],
                osem.at[b],
            ).wait()

        def t_of(kk):
            return wid + _NW * kk

        def body_one(kk, b):
            t = t_of(kk)
            is_tail = jnp.logical_and(wid == _TW, t == _NT - 1)

            @pl.when(kk + 7 < nt)
            def _():
                tn = t_of(kk + 7)

                @pl.when(jnp.logical_or(wid != _TW, tn != _NT - 1))
                def _():
                    in_start(tn, (b + 7) % 8)

                @pl.when(jnp.logical_and(wid == _TW, tn == _NT - 1))
                def _():
                    in_start_tail((b + 7) % 8)

            in_wait(b)

            @pl.when(kk >= 8)
            def _():
                out_wait(b, EMBEDDING_DIM)

            transpose(b, _CHUNK)

            @pl.when(jnp.logical_not(is_tail))
            def _():
                out_start(t, b, EMBEDDING_DIM)

            @pl.when(is_tail)
            def _():
                out_start(t, b, _TAIL * EMBEDDING_DIM // _CHUNK)

        for kk0 in range(7):
            in_start(t_of(kk0), kk0)

        def oct_(k8, carry):
            for bb in range(8):
                kk = 8 * k8 + bb

                @pl.when(kk < nt)
                def _():
                    body_one(kk, bb)
            return carry

        lax.fori_loop(0, (244 + 1 + 7) // 8, oct_, 0)

        @pl.when(wid == _TW)
        def _():
            for b in (5, 6, 7, 0, 1, 2, 3):
                out_wait(b, EMBEDDING_DIM)
            out_wait(4, _TAIL * EMBEDDING_DIM // _CHUNK)

        @pl.when(jnp.logical_and(wid < _NT % _NW, wid != _TW))
        def _():
            for b in (5, 6, 7, 0, 1, 2, 3, 4):
                out_wait(b, EMBEDDING_DIM)

        @pl.when(wid >= _NT % _NW)
        def _():
            for b in (4, 5, 6, 7, 0, 1, 2, 3):
                out_wait(b, EMBEDDING_DIM)

    return k


def _make_gather():
    mesh = plsc.VectorSubcoreMesh(core_axis_name="c", subcore_axis_name="s")

    @functools.partial(
        pl.kernel,
        mesh=mesh,
        out_type=jax.ShapeDtypeStruct(
            (FIELDS, EMBEDDING_DIM // 8, BATCH // _CHUNK, 8, _CHUNK), jnp.float32
        ),
        scratch_types=[
            pltpu.VMEM((_NCH, _CHUNK), jnp.int32),
            pltpu.VMEM((_NB * _CHUNK, EMBEDDING_DIM), jnp.float32),
            pltpu.VMEM((_NB, EMBEDDING_DIM, _CHUNK + 1), jnp.float32),
            pltpu.SemaphoreType.DMA((_NB,)),
            pltpu.SemaphoreType.DMA((_NB,)),
        ],
        compiler_params=pltpu.CompilerParams(
            use_tc_tiling_on_sc=False, needs_layout_passes=False
        ),
    )
    def k(idx_hbm, table_hbm, out_hbm, idx_v, rows_v, tr_v, gsem, osem):
        wid = lax.axis_index("s") * 2 + lax.axis_index("c")
        ch0 = wid * _NCH
        pltpu.sync_copy(idx_hbm.at[wid], idx_v)

        def gather_start(c, b):
            pltpu.async_copy(
                table_hbm.at[idx_v.at[c]],
                rows_v.at[pl.ds(b * _CHUNK, _CHUNK)],
                gsem.at[b],
            )

        def gather_wait(b):
            pltpu.make_async_copy(
                table_hbm.at[idx_v.at[0]],
                rows_v.at[pl.ds(b * _CHUNK, _CHUNK)],
                gsem.at[b],
            ).wait()

        def out_start(j, b):
            ch = ch0 + j
            f = ch >> 7
            bb = ch & 127
            for g in range(4):
                pltpu.async_copy(
                    tr_v.at[b, pl.ds(8 * g, 8), pl.ds(0, _CHUNK)],
                    out_hbm.at[f, g, bb],
                    osem.at[b],
                )

        def out_wait(b):
            for g in range(4):
                pltpu.make_async_copy(
                    tr_v.at[b, pl.ds(8 * g, 8), pl.ds(0, _CHUNK)],
                    out_hbm.at[0, g, 0],
                    osem.at[b],
                ).wait()

        lane = lax.iota(jnp.int32, 16)
        chalf = [lane, lane + 16]

        for c in range(_LEAD):
            gather_start(c, c)

        def group(gi, carry):
            g0 = gi * _NB
            for b in range(_NB):
                j = g0 + b
                nxt = j + _LEAD

                @pl.when(nxt < _NCH)
                def _():
                    gather_start(nxt, (b + _LEAD) % _NB)

                gather_wait(b)

                @pl.when(j >= _NB)
                def _():
                    out_wait(b)

                bsp = jnp.full((16,), b, jnp.int32)

                @plsc.parallel_loop(0, _CHUNK, unroll=8)
                def _tr(r):
                    rsp = jnp.full((16,), r, jnp.int32)
                    for h in range(2):
                        v = rows_v[b * _CHUNK + r, pl.ds(16 * h, 16)]
                        plsc.store_scatter(tr_v, [bsp, chalf[h], rsp], v)

                out_start(j, b)
            return carry

        lax.fori_loop(0, _NCH // _NB, group, 0)
        for b in range(_NB):
            out_wait(b)

    return k


@jax.jit
def kernel(input, weight):
    idx = input.T.reshape(_NW, _NCH, _CHUNK)
    wt = weight.T
    wtail = jnp.pad(wt[:, _CHUNK * (_NT - 1):], ((0, 0), (0, _CHUNK - _TAIL)))
    wlin = _make_fmt()(wt, wtail)
    table = wlin.reshape(NUM_EMBEDDINGS, EMBEDDING_DIM)
    out5d = _make_gather()(idx, table)
    return out5d.transpose(2, 4, 0, 1, 3).reshape(BATCH, FIELDS, EMBEDDING_DIM)
